# Initial kernel scaffold; baseline (speedup 1.0000x reference)
import jax
import jax.numpy as jnp
from jax.experimental import pallas as pl


def kernel(x, edge_index, batch, W1_0, W1_rest, b1, W2, b2, bng1, bnb1, bng2, bnb2, bng3, bnb3, P0, Pb0, P_rest, Pb_rest):
    raise NotImplementedError("write your pallas kernel here")


# speedup vs baseline: 5.5872x; 5.5872x over previous
import functools

import jax
import jax.numpy as jnp
from jax import lax
from jax.experimental import pallas as pl
from jax.experimental.pallas import tpu as pltpu
from jax.experimental.pallas import tpu_sc as plsc

N = 100000
E = 1600000
DIN = 33
H = 32
G = 512
NLAYERS = 5
BN_EPS = 1e-5

NC = 2
NS = 16
HALF = N // NC
ACC_ROWS = 50176
ZROWS = ACC_ROWS // NS
SUB = 4
CHUNK_E = SUB * 128
CHUNKS_PER_TILE = -(-E // (NS * CHUNK_E))
E_PAD = NS * CHUNK_E * CHUNKS_PER_TILE
ROWS_PER_TILE = CHUNKS_PER_TILE * SUB

RB = 2000
NB = N // RB
HALF_BLOCKS = HALF // RB


def _sc_agg_body(y_hbm, src_hbm, dst_hbm, zero_hbm, agg_hbm,
                 acc, src_v, dst_v, lidx_v, rows_v):
  c = lax.axis_index("c")
  s = lax.axis_index("s")
  pltpu.sync_copy(zero_hbm, acc.at[pl.ds(s * ZROWS, ZROWS)])
  plsc.subcore_barrier()

  base_row = s * ROWS_PER_TILE
  coff = c * HALF

  def body(k, carry):
    row0 = base_row + k * SUB
    pltpu.sync_copy(src_hbm.at[pl.ds(row0, SUB)], src_v)
    pltpu.sync_copy(dst_hbm.at[pl.ds(row0, SUB)], dst_v)
    for j in range(SUB):
      pltpu.sync_copy(y_hbm.at[src_v.at[j]], rows_v.at[j])
    for j in range(SUB):
      for i in range(128 // 16):
        v = dst_v[j, pl.ds(i * 16, 16)]
        lv = v - coff
        lv = jnp.where((lv < 0) | (lv >= HALF), HALF, lv)
        lidx_v[j, pl.ds(i * 16, 16)] = lv
    for j in range(SUB):
      pltpu.sync_copy(rows_v.at[j], acc.at[lidx_v.at[j]], add=True)
    return carry

  lax.fori_loop(0, CHUNKS_PER_TILE, body, 0)
  plsc.subcore_barrier()
  pltpu.sync_copy(acc.at[pl.ds(s * ZROWS, ZROWS)],
                  agg_hbm.at[c, pl.ds(s * ZROWS, ZROWS)])


@functools.lru_cache(maxsize=1)
def _get_sc_agg():
  return pl.kernel(
      _sc_agg_body,
      out_type=jax.ShapeDtypeStruct((NC, ACC_ROWS, H), jnp.float32),
      mesh=plsc.VectorSubcoreMesh(core_axis_name="c", subcore_axis_name="s"),
      compiler_params=pltpu.CompilerParams(use_tc_tiling_on_sc=False),
      scratch_types=[
          pltpu.VMEM_SHARED((ACC_ROWS, H), jnp.float32),
          pltpu.VMEM((SUB, 128), jnp.int32),
          pltpu.VMEM((SUB, 128), jnp.int32),
          pltpu.VMEM((SUB, 128), jnp.int32),
          pltpu.VMEM((SUB, 128, H), jnp.float32),
      ],
  )


def _onehot_pool(ids, rows):
  oh = (lax.broadcasted_iota(jnp.int32, (RB, G), 1) == ids[:, None])
  return lax.dot_general(oh.astype(jnp.float32), rows,
                         (((0,), (0,)), ((), ())),
                         preferred_element_type=jnp.float32)


def _tc0_body(x_ref, b3_ref, w1_ref, p0_ref, y_ref, pool_ref, pacc):
  i = pl.program_id(0)
  x = x_ref[...]
  y_ref[...] = jnp.dot(x, w1_ref[...], preferred_element_type=jnp.float32)
  xp = jnp.dot(x, p0_ref[...], preferred_element_type=jnp.float32)

  @pl.when(i == 0)
  def _():
    pacc[...] = jnp.zeros_like(pacc)

  pacc[...] += _onehot_pool(b3_ref[0, 0, :], xp)

  @pl.when(i == NB - 1)
  def _():
    pool_ref[...] = pacc[...]


def _tc_mlp_body(y_ref, agg_ref, b3_ref, a1, d1, w2, a2, d2, a3, c3, wn,
                 yout_ref, pool_ref, pacc, *, has_next):
  i = pl.program_id(0)
  t = y_ref[...] + agg_ref[0]
  u1 = jnp.maximum(t * a1[...] + d1[...], 0.0)
  z2 = jnp.dot(u1, w2[...], preferred_element_type=jnp.float32)
  u2 = jnp.maximum(z2 * a2[...] + d2[...], 0.0)
  h = jnp.maximum(u2 * a3[...] + c3[...], 0.0)
  if has_next:
    yout_ref[...] = jnp.dot(h, wn[...], preferred_element_type=jnp.float32)

  @pl.when(i == 0)
  def _():
    pacc[...] = jnp.zeros_like(pacc)

  pacc[...] += _onehot_pool(b3_ref[0, 0, :], h)

  @pl.when(i == NB - 1)
  def _():
    pool_ref[...] = pacc[...]


def _tc_final_body(pxp_ref, p0r, p1r, p2r, p3r, p4r, prest_ref, cvec_ref,
                   score_ref):
  acc = pxp_ref[...] + cvec_ref[...]
  for l, pr in enumerate((p0r, p1r, p2r, p3r, p4r)):
    acc = acc + jnp.dot(pr[...], prest_ref[l],
                        preferred_element_type=jnp.float32)
  score_ref[...] = acc


def _row_spec(w):
  return pl.BlockSpec((RB, w), lambda i: (i, 0))


def _full2(shape):
  return pl.BlockSpec(shape, lambda i: tuple(0 for _ in shape))


_B3_SPEC = pl.BlockSpec((1, 1, RB), lambda i: (i, 0, 0))
_POOL_SPEC = pl.BlockSpec((G, H), lambda i: (0, 0))
_AGG_SPEC = pl.BlockSpec((1, RB, H),
                         lambda i: (i // HALF_BLOCKS, i % HALF_BLOCKS, 0))

_tc0 = pl.pallas_call(
    _tc0_body,
    grid=(NB,),
    in_specs=[_row_spec(DIN), _B3_SPEC, _full2((DIN, H)), _full2((DIN, H))],
    out_specs=[_row_spec(H), _POOL_SPEC],
    out_shape=[jax.ShapeDtypeStruct((N, H), jnp.float32),
               jax.ShapeDtypeStruct((G, H), jnp.float32)],
    scratch_shapes=[pltpu.VMEM((G, H), jnp.float32)],
)

_VEC_SPECS = [_full2((1, H))] * 2 + [_full2((H, H))] + [_full2((1, H))] * 4

_tc_mlp = pl.pallas_call(
    functools.partial(_tc_mlp_body, has_next=True),
    grid=(NB,),
    in_specs=[_row_spec(H), _AGG_SPEC, _B3_SPEC] + _VEC_SPECS
             + [_full2((H, H))],
    out_specs=[_row_spec(H), _POOL_SPEC],
    out_shape=[jax.ShapeDtypeStruct((N, H), jnp.float32),
               jax.ShapeDtypeStruct((G, H), jnp.float32)],
    scratch_shapes=[pltpu.VMEM((G, H), jnp.float32)],
)

_tc_last = pl.pallas_call(
    functools.partial(_tc_mlp_body, has_next=False),
    grid=(NB,),
    in_specs=[_row_spec(H), _AGG_SPEC, _B3_SPEC] + _VEC_SPECS
             + [_full2((H, H))],
    out_specs=[_row_spec(H), _POOL_SPEC],
    out_shape=[jax.ShapeDtypeStruct((N, H), jnp.float32),
               jax.ShapeDtypeStruct((G, H), jnp.float32)],
    scratch_shapes=[pltpu.VMEM((G, H), jnp.float32)],
)

_tc_final = pl.pallas_call(
    _tc_final_body,
    out_shape=jax.ShapeDtypeStruct((G, H), jnp.float32),
)


def kernel(x, edge_index, batch, W1_0, W1_rest, b1, W2, b2, bng1, bnb1,
           bng2, bnb2, bng3, bnb3, P0, Pb0, P_rest, Pb_rest):
  f32 = jnp.float32
  r = 1.0 / jnp.sqrt(jnp.asarray(1.0 + BN_EPS, f32))
  a1 = bng1 * r
  d1 = b1 * a1 + bnb1
  a2 = bng2 * r
  d2 = b2 * a2 + bnb2
  a3 = bng3 * r
  c3 = bnb3

  src = edge_index[0]
  dst = edge_index[1]
  pad = E_PAD - E
  src2d = jnp.concatenate([src, jnp.zeros((pad,), jnp.int32)]).reshape(
      E_PAD // 128, 128)
  dst2d = jnp.concatenate([dst, jnp.full((pad,), N, jnp.int32)]).reshape(
      E_PAD // 128, 128)
  zero_rows = jnp.zeros((ZROWS, H), f32)
  batch3 = batch.reshape(NB, 1, RB)

  y, pxp = _tc0(x, batch3, W1_0, P0)

  sc_agg = _get_sc_agg()
  pooled = []
  for l in range(NLAYERS):
    agg = sc_agg(y, src2d, dst2d, zero_rows)
    vecs = [a1[l].reshape(1, H), d1[l].reshape(1, H), W2[l],
            a2[l].reshape(1, H), d2[l].reshape(1, H),
            a3[l].reshape(1, H), c3[l].reshape(1, H)]
    if l < NLAYERS - 1:
      wn = W1_rest[l]
      y, pool_l = _tc_mlp(y, agg, batch3, *vecs, wn)
    else:
      _, pool_l = _tc_last(y, agg, batch3, *vecs, jnp.zeros((H, H), f32))
    pooled.append(pool_l)

  cvec = (Pb0 + Pb_rest.sum(axis=0)).reshape(1, H)
  score = _tc_final(pxp, *pooled, P_rest, cvec)
  return (score,) + tuple(pooled)


# speedup vs baseline: 5.7817x; 1.0348x over previous
import functools

import jax
import jax.numpy as jnp
from jax import lax
from jax.experimental import pallas as pl
from jax.experimental.pallas import tpu as pltpu
from jax.experimental.pallas import tpu_sc as plsc

N = 100000
E = 1600000
DIN = 33
H = 32
G = 512
NLAYERS = 5
BN_EPS = 1e-5

NC = 2
NS = 16
HALF = N // NC
ACC_ROWS = 50176
ZROWS = ACC_ROWS // NS
SUB = 2
CHUNK_E = SUB * 128
CHUNKS_PER_TILE = 2 * (-(-E // (NS * CHUNK_E * 2)))
E_PAD = NS * CHUNK_E * CHUNKS_PER_TILE
ROWS_PER_TILE = CHUNKS_PER_TILE * SUB

RB = 2000
NB = N // RB
HALF_BLOCKS = HALF // RB


def _sc_agg_body(y_hbm, src_hbm, dst_hbm, zero_hbm, agg_hbm,
                 acc, src_v, dst_v, lidx_v, rows_v,
                 sem_g, sem_i0, sem_i1, sem_s0, sem_s1):
  c = lax.axis_index("c")
  s = lax.axis_index("s")
  sem_i = (sem_i0, sem_i1)
  sem_s = (sem_s0, sem_s1)
  pltpu.sync_copy(zero_hbm, acc.at[pl.ds(s * ZROWS, ZROWS)])
  plsc.subcore_barrier()

  base_row = s * ROWS_PER_TILE
  coff = c * HALF
  npairs = CHUNKS_PER_TILE // 2

  def fire_idx(k, b):
    row0 = base_row + k * SUB
    pltpu.async_copy(src_hbm.at[pl.ds(row0, SUB)], src_v.at[b], sem_i[b])
    pltpu.async_copy(dst_hbm.at[pl.ds(row0, SUB)], dst_v.at[b], sem_i[b])

  def wait_idx(k, b):
    row0 = base_row + k * SUB
    pltpu.make_async_copy(src_hbm.at[pl.ds(row0, SUB)], src_v.at[b],
                          sem_i[b]).wait()
    pltpu.make_async_copy(dst_hbm.at[pl.ds(row0, SUB)], dst_v.at[b],
                          sem_i[b]).wait()

  def wait_scatters(b):
    for j in range(SUB):
      pltpu.make_async_copy(rows_v.at[b, j], acc.at[lidx_v.at[b, j]],
                            sem_s[b]).wait()

  fire_idx(0, 0)
  fire_idx(1, 1)

  def body(p, carry):
    for b in range(2):
      k = 2 * p + b
      wait_idx(k, b)

      @pl.when(p >= 1)
      def _():
        wait_scatters(b)

      gathers = [
          pltpu.async_copy(y_hbm.at[src_v.at[b, j]], rows_v.at[b, j], sem_g)
          for j in range(SUB)
      ]
      for j in range(SUB):
        for i in range(128 // 16):
          v = dst_v[b, j, pl.ds(i * 16, 16)]
          lv = v - coff
          lv = jnp.where((lv < 0) | (lv >= HALF), HALF, lv)
          lidx_v[b, j, pl.ds(i * 16, 16)] = lv
      for g in gathers:
        g.wait()
      for j in range(SUB):
        pltpu.async_copy(rows_v.at[b, j], acc.at[lidx_v.at[b, j]],
                         sem_s[b], add=True)

      @pl.when(p + 1 < npairs)
      def _():
        fire_idx(k + 2, b)
    return carry

  lax.fori_loop(0, npairs, body, 0)
  wait_scatters(0)
  wait_scatters(1)
  plsc.subcore_barrier()
  pltpu.sync_copy(acc.at[pl.ds(s * ZROWS, ZROWS)],
                  agg_hbm.at[c, pl.ds(s * ZROWS, ZROWS)])


@functools.lru_cache(maxsize=1)
def _get_sc_agg():
  return pl.kernel(
      _sc_agg_body,
      out_type=jax.ShapeDtypeStruct((NC, ACC_ROWS, H), jnp.float32),
      mesh=plsc.VectorSubcoreMesh(core_axis_name="c", subcore_axis_name="s"),
      compiler_params=pltpu.CompilerParams(use_tc_tiling_on_sc=False),
      scratch_types=[
          pltpu.VMEM_SHARED((ACC_ROWS, H), jnp.float32),
          pltpu.VMEM((2, SUB, 128), jnp.int32),
          pltpu.VMEM((2, SUB, 128), jnp.int32),
          pltpu.VMEM((2, SUB, 128), jnp.int32),
          pltpu.VMEM((2, SUB, 128, H), jnp.float32),
          pltpu.SemaphoreType.DMA,
          pltpu.SemaphoreType.DMA,
          pltpu.SemaphoreType.DMA,
          pltpu.SemaphoreType.DMA,
          pltpu.SemaphoreType.DMA,
      ],
  )


def _onehot_pool(ids, rows):
  oh = (lax.broadcasted_iota(jnp.int32, (RB, G), 1) == ids[:, None])
  return lax.dot_general(oh.astype(jnp.float32), rows,
                         (((0,), (0,)), ((), ())),
                         preferred_element_type=jnp.float32)


def _tc0_body(x_ref, b3_ref, w1_ref, p0_ref, y_ref, pool_ref, pacc):
  i = pl.program_id(0)
  x = x_ref[...]
  y_ref[...] = jnp.dot(x, w1_ref[...], preferred_element_type=jnp.float32)
  xp = jnp.dot(x, p0_ref[...], preferred_element_type=jnp.float32)

  @pl.when(i == 0)
  def _():
    pacc[...] = jnp.zeros_like(pacc)

  pacc[...] += _onehot_pool(b3_ref[0, 0, :], xp)

  @pl.when(i == NB - 1)
  def _():
    pool_ref[...] = pacc[...]


def _tc_mlp_body(y_ref, agg_ref, b3_ref, a1, d1, w2, a2, d2, a3, c3, wn,
                 yout_ref, pool_ref, pacc, *, has_next):
  i = pl.program_id(0)
  t = y_ref[...] + agg_ref[0]
  u1 = jnp.maximum(t * a1[...] + d1[...], 0.0)
  z2 = jnp.dot(u1, w2[...], preferred_element_type=jnp.float32)
  u2 = jnp.maximum(z2 * a2[...] + d2[...], 0.0)
  h = jnp.maximum(u2 * a3[...] + c3[...], 0.0)
  if has_next:
    yout_ref[...] = jnp.dot(h, wn[...], preferred_element_type=jnp.float32)

  @pl.when(i == 0)
  def _():
    pacc[...] = jnp.zeros_like(pacc)

  pacc[...] += _onehot_pool(b3_ref[0, 0, :], h)

  @pl.when(i == NB - 1)
  def _():
    pool_ref[...] = pacc[...]


def _tc_final_body(pxp_ref, p0r, p1r, p2r, p3r, p4r, prest_ref, cvec_ref,
                   score_ref):
  acc = pxp_ref[...] + cvec_ref[...]
  for l, pr in enumerate((p0r, p1r, p2r, p3r, p4r)):
    acc = acc + jnp.dot(pr[...], prest_ref[l],
                        preferred_element_type=jnp.float32)
  score_ref[...] = acc


def _row_spec(w):
  return pl.BlockSpec((RB, w), lambda i: (i, 0))


def _full2(shape):
  return pl.BlockSpec(shape, lambda i: tuple(0 for _ in shape))


_B3_SPEC = pl.BlockSpec((1, 1, RB), lambda i: (i, 0, 0))
_POOL_SPEC = pl.BlockSpec((G, H), lambda i: (0, 0))
_AGG_SPEC = pl.BlockSpec((1, RB, H),
                         lambda i: (i // HALF_BLOCKS, i % HALF_BLOCKS, 0))

_tc0 = pl.pallas_call(
    _tc0_body,
    grid=(NB,),
    in_specs=[_row_spec(DIN), _B3_SPEC, _full2((DIN, H)), _full2((DIN, H))],
    out_specs=[_row_spec(H), _POOL_SPEC],
    out_shape=[jax.ShapeDtypeStruct((N, H), jnp.float32),
               jax.ShapeDtypeStruct((G, H), jnp.float32)],
    scratch_shapes=[pltpu.VMEM((G, H), jnp.float32)],
)

_VEC_SPECS = [_full2((1, H))] * 2 + [_full2((H, H))] + [_full2((1, H))] * 4

_tc_mlp = pl.pallas_call(
    functools.partial(_tc_mlp_body, has_next=True),
    grid=(NB,),
    in_specs=[_row_spec(H), _AGG_SPEC, _B3_SPEC] + _VEC_SPECS
             + [_full2((H, H))],
    out_specs=[_row_spec(H), _POOL_SPEC],
    out_shape=[jax.ShapeDtypeStruct((N, H), jnp.float32),
               jax.ShapeDtypeStruct((G, H), jnp.float32)],
    scratch_shapes=[pltpu.VMEM((G, H), jnp.float32)],
)

_tc_last = pl.pallas_call(
    functools.partial(_tc_mlp_body, has_next=False),
    grid=(NB,),
    in_specs=[_row_spec(H), _AGG_SPEC, _B3_SPEC] + _VEC_SPECS
             + [_full2((H, H))],
    out_specs=[_row_spec(H), _POOL_SPEC],
    out_shape=[jax.ShapeDtypeStruct((N, H), jnp.float32),
               jax.ShapeDtypeStruct((G, H), jnp.float32)],
    scratch_shapes=[pltpu.VMEM((G, H), jnp.float32)],
)

_tc_final = pl.pallas_call(
    _tc_final_body,
    out_shape=jax.ShapeDtypeStruct((G, H), jnp.float32),
)


def kernel(x, edge_index, batch, W1_0, W1_rest, b1, W2, b2, bng1, bnb1,
           bng2, bnb2, bng3, bnb3, P0, Pb0, P_rest, Pb_rest):
  f32 = jnp.float32
  r = 1.0 / jnp.sqrt(jnp.asarray(1.0 + BN_EPS, f32))
  a1 = bng1 * r
  d1 = b1 * a1 + bnb1
  a2 = bng2 * r
  d2 = b2 * a2 + bnb2
  a3 = bng3 * r
  c3 = bnb3

  src = edge_index[0]
  dst = edge_index[1]
  pad = E_PAD - E
  src2d = jnp.concatenate([src, jnp.zeros((pad,), jnp.int32)]).reshape(
      E_PAD // 128, 128)
  dst2d = jnp.concatenate([dst, jnp.full((pad,), N, jnp.int32)]).reshape(
      E_PAD // 128, 128)
  zero_rows = jnp.zeros((ZROWS, H), f32)
  batch3 = batch.reshape(NB, 1, RB)

  y, pxp = _tc0(x, batch3, W1_0, P0)

  sc_agg = _get_sc_agg()
  pooled = []
  for l in range(NLAYERS):
    agg = sc_agg(y, src2d, dst2d, zero_rows)
    vecs = [a1[l].reshape(1, H), d1[l].reshape(1, H), W2[l],
            a2[l].reshape(1, H), d2[l].reshape(1, H),
            a3[l].reshape(1, H), c3[l].reshape(1, H)]
    if l < NLAYERS - 1:
      wn = W1_rest[l]
      y, pool_l = _tc_mlp(y, agg, batch3, *vecs, wn)
    else:
      _, pool_l = _tc_last(y, agg, batch3, *vecs, jnp.zeros((H, H), f32))
    pooled.append(pool_l)

  cvec = (Pb0 + Pb_rest.sum(axis=0)).reshape(1, H)
  score = _tc_final(pxp, *pooled, P_rest, cvec)
  return (score,) + tuple(pooled)


# speedup vs baseline: 10.6541x; 1.8427x over previous
import functools

import jax
import jax.numpy as jnp
from jax import lax
from jax.experimental import pallas as pl
from jax.experimental.pallas import tpu as pltpu
from jax.experimental.pallas import tpu_sc as plsc

N = 100000
E = 1600000
DIN = 33
H = 32
G = 512
NLAYERS = 5
BN_EPS = 1e-5

NC = 2
NS = 16
HALF = N // NC
ACC_ROWS = 50176
ZROWS = ACC_ROWS // NS
SUB = 4
CHUNK_E = SUB * 128
E_PAD = 1605632
EPR = E_PAD // 128
SHARE_ROWS = EPR // NS
IN_CHUNKS = SHARE_ROWS // 8
FLUSH = 1024
CAPR = NS * (SHARE_ROWS + 8)
PAD_DST = N

RB = 2000
NB = N // RB
HALF_BLOCKS = HALF // RB


def _sc_part_body(src_hbm, dst_hbm, bsrc_hbm, bdst_hbm, cnt_hbm,
                  in_src, in_dst, stage_src, stage_dst,
                  flush_src, flush_dst, cnt_v, cnt_smem, sem_f, sem_ld):
  c = lax.axis_index("c")
  s = lax.axis_index("s")
  coff = c * HALF

  @pl.when(s == 0)
  def _():
    cnt_smem[0] = 0
  plsc.subcore_barrier()

  def wait_flush():
    pltpu.make_async_copy(flush_src, bsrc_hbm.at[c, pl.ds(0, FLUSH)],
                          sem_f).wait()
    pltpu.make_async_copy(flush_dst, bdst_hbm.at[c, pl.ds(0, FLUSH)],
                          sem_f).wait()

  def chunk(k, carry):
    w, flushed = carry
    row0 = s * SHARE_ROWS + k * 8
    pltpu.async_copy(src_hbm.at[pl.ds(row0, 8)], in_src, sem_ld)
    pltpu.async_copy(dst_hbm.at[pl.ds(row0, 8)], in_dst, sem_ld)
    pltpu.make_async_copy(src_hbm.at[pl.ds(row0, 8)], in_src, sem_ld).wait()
    pltpu.make_async_copy(dst_hbm.at[pl.ds(row0, 8)], in_dst, sem_ld).wait()
    for z in range(64):
      zr, zc = z // 8, z % 8
      srcv = in_src[zr, pl.ds(zc * 16, 16)]
      dstv = in_dst[zr, pl.ds(zc * 16, 16)]
      m = (dstv >= coff) & (dstv < coff + HALF)
      nkeep = jnp.max(plsc.all_reduce_population_count(m))
      plsc.store_compressed(stage_src.at[pl.ds(w, 16)], srcv, mask=m)
      plsc.store_compressed(stage_dst.at[pl.ds(w, 16)], dstv, mask=m)
      w = w + nkeep
    do_flush = w >= FLUSH

    @pl.when(do_flush)
    def _():
      @pl.when(flushed == 1)
      def _():
        wait_flush()
      for q in range(FLUSH // 16):
        flush_src[pl.ds(q * 16, 16)] = stage_src[pl.ds(q * 16, 16)]
        flush_dst[pl.ds(q * 16, 16)] = stage_dst[pl.ds(q * 16, 16)]
      off = plsc.fetch_and_add(cnt_smem.at[0], FLUSH, subcore_id=0)
      off = pl.multiple_of(off, FLUSH)
      pltpu.async_copy(flush_src, bsrc_hbm.at[c, pl.ds(off, FLUSH)], sem_f)
      pltpu.async_copy(flush_dst, bdst_hbm.at[c, pl.ds(off, FLUSH)], sem_f)
      for q in range(FLUSH // 16):
        tshift_s = stage_src[pl.ds(FLUSH + q * 16, 16)]
        tshift_d = stage_dst[pl.ds(FLUSH + q * 16, 16)]
        stage_src[pl.ds(q * 16, 16)] = tshift_s
        stage_dst[pl.ds(q * 16, 16)] = tshift_d

    flushed = jnp.where(do_flush, 1, flushed)
    w = jnp.where(do_flush, w - FLUSH, w)
    return (w, flushed)

  w, flushed = lax.fori_loop(0, IN_CHUNKS, chunk, (jnp.int32(0), jnp.int32(0)))

  @pl.when(flushed == 1)
  def _():
    wait_flush()

  @pl.when(w > 0)
  def _():
    for q in range(FLUSH // 16):
      idxv = q * 16 + lax.iota(jnp.int32, 16)
      keep = idxv < w
      tsrc = jnp.where(keep, stage_src[pl.ds(q * 16, 16)], 0)
      tdst = jnp.where(keep, stage_dst[pl.ds(q * 16, 16)], PAD_DST)
      flush_src[pl.ds(q * 16, 16)] = tsrc
      flush_dst[pl.ds(q * 16, 16)] = tdst
    off = plsc.fetch_and_add(cnt_smem.at[0], FLUSH, subcore_id=0)
    off = pl.multiple_of(off, FLUSH)
    pltpu.async_copy(flush_src, bsrc_hbm.at[c, pl.ds(off, FLUSH)], sem_f)
    pltpu.async_copy(flush_dst, bdst_hbm.at[c, pl.ds(off, FLUSH)], sem_f)
    wait_flush()

  plsc.subcore_barrier()

  @pl.when(s == 0)
  def _():
    total = cnt_smem[0]
    cnt_v[pl.ds(0, 16)] = jnp.full((16,), total, jnp.int32)
    pltpu.async_copy(cnt_v, cnt_hbm.at[c], sem_f)
    pltpu.make_async_copy(cnt_v, cnt_hbm.at[c], sem_f).wait()


@functools.lru_cache(maxsize=1)
def _get_sc_part():
  return pl.kernel(
      _sc_part_body,
      out_type=(
          jax.ShapeDtypeStruct((NC, CAPR * 128), jnp.int32),
          jax.ShapeDtypeStruct((NC, CAPR * 128), jnp.int32),
          jax.ShapeDtypeStruct((NC, 16), jnp.int32),
      ),
      mesh=plsc.VectorSubcoreMesh(core_axis_name="c", subcore_axis_name="s"),
      compiler_params=pltpu.CompilerParams(use_tc_tiling_on_sc=False,
                                           needs_layout_passes=False),
      scratch_types=[
          pltpu.VMEM((8, 128), jnp.int32),
          pltpu.VMEM((8, 128), jnp.int32),
          pltpu.VMEM((2 * FLUSH + 16,), jnp.int32),
          pltpu.VMEM((2 * FLUSH + 16,), jnp.int32),
          pltpu.VMEM((FLUSH,), jnp.int32),
          pltpu.VMEM((FLUSH,), jnp.int32),
          pltpu.VMEM((16,), jnp.int32),
          pltpu.SMEM((8,), jnp.int32),
          pltpu.SemaphoreType.DMA,
          pltpu.SemaphoreType.DMA,
      ],
  )


def _sc_agg_body(y_hbm, bsrc_hbm, bdst_hbm, cnt_hbm, zero_hbm, agg_hbm,
                 acc, src_v, dst_v, lidx_v, rows_v, cnt_v, sem_g, sem_s):
  c = lax.axis_index("c")
  s = lax.axis_index("s")
  pltpu.sync_copy(cnt_hbm.at[c], cnt_v)
  pltpu.sync_copy(zero_hbm, acc.at[pl.ds(s * ZROWS, ZROWS)])
  n_edges = jnp.max(cnt_v[pl.ds(0, 16)])
  n_chunks = n_edges // CHUNK_E
  trips = jnp.maximum((n_chunks - s + NS - 1) // NS, 0)
  plsc.subcore_barrier()
  coff = c * HALF

  def wait_scatters():
    for j in range(SUB):
      pltpu.make_async_copy(rows_v.at[j], acc.at[lidx_v.at[j]],
                            sem_s).wait()

  def body(i, carry):
    @pl.when(i >= 1)
    def _():
      wait_scatters()
    row0 = (s + i * NS) * SUB
    pltpu.sync_copy(bsrc_hbm.at[c, pl.ds(row0, SUB)], src_v)
    pltpu.sync_copy(bdst_hbm.at[c, pl.ds(row0, SUB)], dst_v)
    gathers = [
        pltpu.async_copy(y_hbm.at[src_v.at[j]], rows_v.at[j], sem_g)
        for j in range(SUB)
    ]
    for j in range(SUB):
      for i2 in range(128 // 16):
        v = dst_v[j, pl.ds(i2 * 16, 16)]
        lv = v - coff
        lv = jnp.where((lv < 0) | (lv >= HALF), HALF, lv)
        lidx_v[j, pl.ds(i2 * 16, 16)] = lv
    for g in gathers:
      g.wait()
    for j in range(SUB):
      pltpu.async_copy(rows_v.at[j], acc.at[lidx_v.at[j]], sem_s, add=True)
    return carry

  lax.fori_loop(0, trips, body, 0)

  @pl.when(trips >= 1)
  def _():
    wait_scatters()
  plsc.subcore_barrier()
  pltpu.sync_copy(acc.at[pl.ds(s * ZROWS, ZROWS)],
                  agg_hbm.at[c, pl.ds(s * ZROWS, ZROWS)])


@functools.lru_cache(maxsize=1)
def _get_sc_agg():
  return pl.kernel(
      _sc_agg_body,
      out_type=jax.ShapeDtypeStruct((NC, ACC_ROWS, H), jnp.float32),
      mesh=plsc.VectorSubcoreMesh(core_axis_name="c", subcore_axis_name="s"),
      compiler_params=pltpu.CompilerParams(use_tc_tiling_on_sc=False,
                                           needs_layout_passes=False),
      scratch_types=[
          pltpu.VMEM_SHARED((ACC_ROWS, H), jnp.float32),
          pltpu.VMEM((SUB, 128), jnp.int32),
          pltpu.VMEM((SUB, 128), jnp.int32),
          pltpu.VMEM((SUB, 128), jnp.int32),
          pltpu.VMEM((SUB, 128, H), jnp.float32),
          pltpu.VMEM((16,), jnp.int32),
          pltpu.SemaphoreType.DMA,
          pltpu.SemaphoreType.DMA,
      ],
  )


def _onehot_pool(ids, rows):
  oh = (lax.broadcasted_iota(jnp.int32, (RB, G), 1) == ids[:, None])
  return lax.dot_general(oh.astype(jnp.float32), rows,
                         (((0,), (0,)), ((), ())),
                         preferred_element_type=jnp.float32)


def _tc0_body(x_ref, b3_ref, w1_ref, p0_ref, y_ref, pool_ref, pacc):
  i = pl.program_id(0)
  x = x_ref[...]
  y_ref[...] = jnp.dot(x, w1_ref[...], preferred_element_type=jnp.float32)
  xp = jnp.dot(x, p0_ref[...], preferred_element_type=jnp.float32)

  @pl.when(i == 0)
  def _():
    pacc[...] = jnp.zeros_like(pacc)

  pacc[...] += _onehot_pool(b3_ref[0, 0, :], xp)

  @pl.when(i == NB - 1)
  def _():
    pool_ref[...] = pacc[...]


def _tc_mlp_body(y_ref, agg_ref, b3_ref, a1, d1, w2, a2, d2, a3, c3, wn,
                 yout_ref, pool_ref, pacc, *, has_next):
  i = pl.program_id(0)
  t = y_ref[...] + agg_ref[0]
  u1 = jnp.maximum(t * a1[...] + d1[...], 0.0)
  z2 = jnp.dot(u1, w2[...], preferred_element_type=jnp.float32)
  u2 = jnp.maximum(z2 * a2[...] + d2[...], 0.0)
  h = jnp.maximum(u2 * a3[...] + c3[...], 0.0)
  if has_next:
    yout_ref[...] = jnp.dot(h, wn[...], preferred_element_type=jnp.float32)

  @pl.when(i == 0)
  def _():
    pacc[...] = jnp.zeros_like(pacc)

  pacc[...] += _onehot_pool(b3_ref[0, 0, :], h)

  @pl.when(i == NB - 1)
  def _():
    pool_ref[...] = pacc[...]


def _tc_final_body(pxp_ref, p0r, p1r, p2r, p3r, p4r, prest_ref, cvec_ref,
                   score_ref):
  acc = pxp_ref[...] + cvec_ref[...]
  for l, pr in enumerate((p0r, p1r, p2r, p3r, p4r)):
    acc = acc + jnp.dot(pr[...], prest_ref[l],
                        preferred_element_type=jnp.float32)
  score_ref[...] = acc


def _row_spec(w):
  return pl.BlockSpec((RB, w), lambda i: (i, 0))


def _full2(shape):
  return pl.BlockSpec(shape, lambda i: tuple(0 for _ in shape))


_B3_SPEC = pl.BlockSpec((1, 1, RB), lambda i: (i, 0, 0))
_POOL_SPEC = pl.BlockSpec((G, H), lambda i: (0, 0))
_AGG_SPEC = pl.BlockSpec((1, RB, H),
                         lambda i: (i // HALF_BLOCKS, i % HALF_BLOCKS, 0))

_tc0 = pl.pallas_call(
    _tc0_body,
    grid=(NB,),
    in_specs=[_row_spec(DIN), _B3_SPEC, _full2((DIN, H)), _full2((DIN, H))],
    out_specs=[_row_spec(H), _POOL_SPEC],
    out_shape=[jax.ShapeDtypeStruct((N, H), jnp.float32),
               jax.ShapeDtypeStruct((G, H), jnp.float32)],
    scratch_shapes=[pltpu.VMEM((G, H), jnp.float32)],
)

_VEC_SPECS = [_full2((1, H))] * 2 + [_full2((H, H))] + [_full2((1, H))] * 4

_tc_mlp = pl.pallas_call(
    functools.partial(_tc_mlp_body, has_next=True),
    grid=(NB,),
    in_specs=[_row_spec(H), _AGG_SPEC, _B3_SPEC] + _VEC_SPECS
             + [_full2((H, H))],
    out_specs=[_row_spec(H), _POOL_SPEC],
    out_shape=[jax.ShapeDtypeStruct((N, H), jnp.float32),
               jax.ShapeDtypeStruct((G, H), jnp.float32)],
    scratch_shapes=[pltpu.VMEM((G, H), jnp.float32)],
)

_tc_last = pl.pallas_call(
    functools.partial(_tc_mlp_body, has_next=False),
    grid=(NB,),
    in_specs=[_row_spec(H), _AGG_SPEC, _B3_SPEC] + _VEC_SPECS
             + [_full2((H, H))],
    out_specs=[_row_spec(H), _POOL_SPEC],
    out_shape=[jax.ShapeDtypeStruct((N, H), jnp.float32),
               jax.ShapeDtypeStruct((G, H), jnp.float32)],
    scratch_shapes=[pltpu.VMEM((G, H), jnp.float32)],
)

_tc_final = pl.pallas_call(
    _tc_final_body,
    out_shape=jax.ShapeDtypeStruct((G, H), jnp.float32),
)


def kernel(x, edge_index, batch, W1_0, W1_rest, b1, W2, b2, bng1, bnb1,
           bng2, bnb2, bng3, bnb3, P0, Pb0, P_rest, Pb_rest):
  f32 = jnp.float32
  r = 1.0 / jnp.sqrt(jnp.asarray(1.0 + BN_EPS, f32))
  a1 = bng1 * r
  d1 = b1 * a1 + bnb1
  a2 = bng2 * r
  d2 = b2 * a2 + bnb2
  a3 = bng3 * r
  c3 = bnb3

  src = edge_index[0]
  dst = edge_index[1]
  pad = E_PAD - E
  src2d = jnp.concatenate([src, jnp.zeros((pad,), jnp.int32)]).reshape(
      E_PAD // 128, 128)
  dst2d = jnp.concatenate([dst, jnp.full((pad,), N, jnp.int32)]).reshape(
      E_PAD // 128, 128)
  zero_rows = jnp.zeros((ZROWS, H), f32)
  batch3 = batch.reshape(NB, 1, RB)

  y, pxp = _tc0(x, batch3, W1_0, P0)

  bsrc_f, bdst_f, cnt = _get_sc_part()(src2d, dst2d)
  bsrc = bsrc_f.reshape(NC, CAPR, 128)
  bdst = bdst_f.reshape(NC, CAPR, 128)

  sc_agg = _get_sc_agg()
  pooled = []
  for l in range(NLAYERS):
    agg = sc_agg(y, bsrc, bdst, cnt, zero_rows)
    vecs = [a1[l].reshape(1, H), d1[l].reshape(1, H), W2[l],
            a2[l].reshape(1, H), d2[l].reshape(1, H),
            a3[l].reshape(1, H), c3[l].reshape(1, H)]
    if l < NLAYERS - 1:
      wn = W1_rest[l]
      y, pool_l = _tc_mlp(y, agg, batch3, *vecs, wn)
    else:
      _, pool_l = _tc_last(y, agg, batch3, *vecs, jnp.zeros((H, H), f32))
    pooled.append(pool_l)

  cvec = (Pb0 + Pb_rest.sum(axis=0)).reshape(1, H)
  score = _tc_final(pxp, *pooled, P_rest, cvec)
  return (score,) + tuple(pooled)


# speedup vs baseline: 12.3241x; 1.1567x over previous
import functools

import jax
import jax.numpy as jnp
from jax import lax
from jax.experimental import pallas as pl
from jax.experimental.pallas import tpu as pltpu
from jax.experimental.pallas import tpu_sc as plsc

N = 100000
E = 1600000
DIN = 33
H = 32
G = 512
NLAYERS = 5
BN_EPS = 1e-5

NC = 2
NS = 16
HALF = N // NC
ACC_ROWS = 50176
ZROWS = ACC_ROWS // NS
SUB = 4
CHUNK_E = SUB * 128
E_PAD = 1605632
EPR = E_PAD // 128
SHARE_ROWS = EPR // NS
IN_CHUNKS = SHARE_ROWS // 8
FLUSH = 1024
CAPR = NS * (SHARE_ROWS + 8)
PAD_DST = N

RB = 2000
NB = N // RB
HALF_BLOCKS = HALF // RB


def _sc_part_body(src_hbm, dst_hbm, bsrc_hbm, bdst_hbm, cnt_hbm,
                  in_src, in_dst, stage_src, stage_dst,
                  flush_src, flush_dst, cnt_v, cnt_smem, sem_f, sem_ld):
  c = lax.axis_index("c")
  s = lax.axis_index("s")
  coff = c * HALF

  @pl.when(s == 0)
  def _():
    cnt_smem[0] = 0
  plsc.subcore_barrier()

  def wait_flush():
    pltpu.make_async_copy(flush_src, bsrc_hbm.at[c, pl.ds(0, FLUSH)],
                          sem_f).wait()
    pltpu.make_async_copy(flush_dst, bdst_hbm.at[c, pl.ds(0, FLUSH)],
                          sem_f).wait()

  def chunk(k, carry):
    w, flushed = carry
    row0 = s * SHARE_ROWS + k * 8
    pltpu.async_copy(src_hbm.at[pl.ds(row0, 8)], in_src, sem_ld)
    pltpu.async_copy(dst_hbm.at[pl.ds(row0, 8)], in_dst, sem_ld)
    pltpu.make_async_copy(src_hbm.at[pl.ds(row0, 8)], in_src, sem_ld).wait()
    pltpu.make_async_copy(dst_hbm.at[pl.ds(row0, 8)], in_dst, sem_ld).wait()
    for z in range(64):
      zr, zc = z // 8, z % 8
      srcv = in_src[zr, pl.ds(zc * 16, 16)]
      dstv = in_dst[zr, pl.ds(zc * 16, 16)]
      m = (dstv >= coff) & (dstv < coff + HALF)
      nkeep = jnp.max(plsc.all_reduce_population_count(m))
      plsc.store_compressed(stage_src.at[pl.ds(w, 16)], srcv, mask=m)
      plsc.store_compressed(stage_dst.at[pl.ds(w, 16)], dstv, mask=m)
      w = w + nkeep
    do_flush = w >= FLUSH

    @pl.when(do_flush)
    def _():
      @pl.when(flushed == 1)
      def _():
        wait_flush()
      for q in range(FLUSH // 16):
        flush_src[pl.ds(q * 16, 16)] = stage_src[pl.ds(q * 16, 16)]
        flush_dst[pl.ds(q * 16, 16)] = stage_dst[pl.ds(q * 16, 16)]
      off = plsc.fetch_and_add(cnt_smem.at[0], FLUSH, subcore_id=0)
      off = pl.multiple_of(off, FLUSH)
      pltpu.async_copy(flush_src, bsrc_hbm.at[c, pl.ds(off, FLUSH)], sem_f)
      pltpu.async_copy(flush_dst, bdst_hbm.at[c, pl.ds(off, FLUSH)], sem_f)
      for q in range(FLUSH // 16):
        tshift_s = stage_src[pl.ds(FLUSH + q * 16, 16)]
        tshift_d = stage_dst[pl.ds(FLUSH + q * 16, 16)]
        stage_src[pl.ds(q * 16, 16)] = tshift_s
        stage_dst[pl.ds(q * 16, 16)] = tshift_d

    flushed = jnp.where(do_flush, 1, flushed)
    w = jnp.where(do_flush, w - FLUSH, w)
    return (w, flushed)

  w, flushed = lax.fori_loop(0, IN_CHUNKS, chunk, (jnp.int32(0), jnp.int32(0)))

  @pl.when(flushed == 1)
  def _():
    wait_flush()

  @pl.when(w > 0)
  def _():
    for q in range(FLUSH // 16):
      idxv = q * 16 + lax.iota(jnp.int32, 16)
      keep = idxv < w
      tsrc = jnp.where(keep, stage_src[pl.ds(q * 16, 16)], 0)
      tdst = jnp.where(keep, stage_dst[pl.ds(q * 16, 16)], PAD_DST)
      flush_src[pl.ds(q * 16, 16)] = tsrc
      flush_dst[pl.ds(q * 16, 16)] = tdst
    off = plsc.fetch_and_add(cnt_smem.at[0], FLUSH, subcore_id=0)
    off = pl.multiple_of(off, FLUSH)
    pltpu.async_copy(flush_src, bsrc_hbm.at[c, pl.ds(off, FLUSH)], sem_f)
    pltpu.async_copy(flush_dst, bdst_hbm.at[c, pl.ds(off, FLUSH)], sem_f)
    wait_flush()

  plsc.subcore_barrier()

  @pl.when(s == 0)
  def _():
    total = cnt_smem[0]
    cnt_v[pl.ds(0, 16)] = jnp.full((16,), total, jnp.int32)
    pltpu.async_copy(cnt_v, cnt_hbm.at[c], sem_f)
    pltpu.make_async_copy(cnt_v, cnt_hbm.at[c], sem_f).wait()


@functools.lru_cache(maxsize=1)
def _get_sc_part():
  return pl.kernel(
      _sc_part_body,
      out_type=(
          jax.ShapeDtypeStruct((NC, CAPR * 128), jnp.int32),
          jax.ShapeDtypeStruct((NC, CAPR * 128), jnp.int32),
          jax.ShapeDtypeStruct((NC, 16), jnp.int32),
      ),
      mesh=plsc.VectorSubcoreMesh(core_axis_name="c", subcore_axis_name="s"),
      compiler_params=pltpu.CompilerParams(use_tc_tiling_on_sc=False,
                                           needs_layout_passes=False),
      scratch_types=[
          pltpu.VMEM((8, 128), jnp.int32),
          pltpu.VMEM((8, 128), jnp.int32),
          pltpu.VMEM((2 * FLUSH + 16,), jnp.int32),
          pltpu.VMEM((2 * FLUSH + 16,), jnp.int32),
          pltpu.VMEM((FLUSH,), jnp.int32),
          pltpu.VMEM((FLUSH,), jnp.int32),
          pltpu.VMEM((16,), jnp.int32),
          pltpu.SMEM((8,), jnp.int32),
          pltpu.SemaphoreType.DMA,
          pltpu.SemaphoreType.DMA,
      ],
  )


def _sc_agg_body(y_hbm, bsrc_hbm, bdst_hbm, cnt_hbm, zero_hbm, agg_hbm,
                 acc, src_v, dst_v, lidx_v, rows_v, cnt_v,
                 sem_g, sem_s, sem_i):
  c = lax.axis_index("c")
  s = lax.axis_index("s")
  pltpu.sync_copy(cnt_hbm.at[c], cnt_v)
  pltpu.sync_copy(zero_hbm, acc.at[pl.ds(s * ZROWS, ZROWS)])
  n_edges = jnp.max(cnt_v[pl.ds(0, 16)])
  n_chunks = n_edges // CHUNK_E
  trips = jnp.maximum((n_chunks - s + NS - 1) // NS, 0)
  plsc.subcore_barrier()
  coff = c * HALF

  def wait_scatters():
    for j in range(SUB):
      pltpu.make_async_copy(rows_v.at[j], acc.at[lidx_v.at[j]],
                            sem_s).wait()

  def fire_idx(i):
    row0 = (s + i * NS) * SUB
    pltpu.async_copy(bsrc_hbm.at[c, pl.ds(row0, SUB)], src_v, sem_i)
    pltpu.async_copy(bdst_hbm.at[c, pl.ds(row0, SUB)], dst_v, sem_i)

  def wait_idx(i):
    row0 = (s + i * NS) * SUB
    pltpu.make_async_copy(bsrc_hbm.at[c, pl.ds(row0, SUB)], src_v,
                          sem_i).wait()
    pltpu.make_async_copy(bdst_hbm.at[c, pl.ds(row0, SUB)], dst_v,
                          sem_i).wait()

  @pl.when(trips >= 1)
  def _():
    fire_idx(0)

  def body(i, carry):
    @pl.when(i >= 1)
    def _():
      wait_scatters()
    wait_idx(i)
    gathers = [
        pltpu.async_copy(y_hbm.at[src_v.at[j]], rows_v.at[j], sem_g)
        for j in range(SUB)
    ]
    for j in range(SUB):
      for i2 in range(128 // 16):
        v = dst_v[j, pl.ds(i2 * 16, 16)]
        lv = v - coff
        lv = jnp.where((lv < 0) | (lv >= HALF), HALF, lv)
        lidx_v[j, pl.ds(i2 * 16, 16)] = lv
    for g in gathers:
      g.wait()
    for j in range(SUB):
      pltpu.async_copy(rows_v.at[j], acc.at[lidx_v.at[j]], sem_s, add=True)

    @pl.when(i + 1 < trips)
    def _():
      fire_idx(i + 1)
    return carry

  lax.fori_loop(0, trips, body, 0)

  @pl.when(trips >= 1)
  def _():
    wait_scatters()
  plsc.subcore_barrier()
  pltpu.sync_copy(acc.at[pl.ds(s * ZROWS, ZROWS)],
                  agg_hbm.at[c, pl.ds(s * ZROWS, ZROWS)])


@functools.lru_cache(maxsize=1)
def _get_sc_agg():
  return pl.kernel(
      _sc_agg_body,
      out_type=jax.ShapeDtypeStruct((NC, ACC_ROWS, H), jnp.float32),
      mesh=plsc.VectorSubcoreMesh(core_axis_name="c", subcore_axis_name="s"),
      compiler_params=pltpu.CompilerParams(use_tc_tiling_on_sc=False,
                                           needs_layout_passes=False),
      scratch_types=[
          pltpu.VMEM_SHARED((ACC_ROWS, H), jnp.float32),
          pltpu.VMEM((SUB, 128), jnp.int32),
          pltpu.VMEM((SUB, 128), jnp.int32),
          pltpu.VMEM((SUB, 128), jnp.int32),
          pltpu.VMEM((SUB, 128, H), jnp.float32),
          pltpu.VMEM((16,), jnp.int32),
          pltpu.SemaphoreType.DMA,
          pltpu.SemaphoreType.DMA,
          pltpu.SemaphoreType.DMA,
      ],
  )


def _onehot_pool(ids, rows):
  oh = (lax.broadcasted_iota(jnp.int32, (RB, G), 1) == ids[:, None])
  return lax.dot_general(oh.astype(jnp.float32), rows,
                         (((0,), (0,)), ((), ())),
                         preferred_element_type=jnp.float32)


def _tc0_body(x_ref, b3_ref, w1_ref, p0_ref, y_ref, pool_ref, pacc):
  i = pl.program_id(0)
  x = x_ref[...]
  y_ref[...] = jnp.dot(x, w1_ref[...], preferred_element_type=jnp.float32)
  xp = jnp.dot(x, p0_ref[...], preferred_element_type=jnp.float32)

  @pl.when(i == 0)
  def _():
    pacc[...] = jnp.zeros_like(pacc)

  pacc[...] += _onehot_pool(b3_ref[0, 0, :], xp)

  @pl.when(i == NB - 1)
  def _():
    pool_ref[...] = pacc[...]


def _tc_mlp_body(y_ref, agg_ref, b3_ref, a1, d1, w2, a2, d2, a3, c3, wn,
                 yout_ref, pool_ref, pacc, *, has_next):
  i = pl.program_id(0)
  t = y_ref[...] + agg_ref[0]
  u1 = jnp.maximum(t * a1[...] + d1[...], 0.0)
  z2 = jnp.dot(u1, w2[...], preferred_element_type=jnp.float32)
  u2 = jnp.maximum(z2 * a2[...] + d2[...], 0.0)
  h = jnp.maximum(u2 * a3[...] + c3[...], 0.0)
  if has_next:
    yout_ref[...] = jnp.dot(h, wn[...], preferred_element_type=jnp.float32)

  @pl.when(i == 0)
  def _():
    pacc[...] = jnp.zeros_like(pacc)

  pacc[...] += _onehot_pool(b3_ref[0, 0, :], h)

  @pl.when(i == NB - 1)
  def _():
    pool_ref[...] = pacc[...]


def _tc_final_body(pxp_ref, p0r, p1r, p2r, p3r, p4r, prest_ref, cvec_ref,
                   score_ref):
  acc = pxp_ref[...] + cvec_ref[...]
  for l, pr in enumerate((p0r, p1r, p2r, p3r, p4r)):
    acc = acc + jnp.dot(pr[...], prest_ref[l],
                        preferred_element_type=jnp.float32)
  score_ref[...] = acc


def _row_spec(w):
  return pl.BlockSpec((RB, w), lambda i: (i, 0))


def _full2(shape):
  return pl.BlockSpec(shape, lambda i: tuple(0 for _ in shape))


_B3_SPEC = pl.BlockSpec((1, 1, RB), lambda i: (i, 0, 0))
_POOL_SPEC = pl.BlockSpec((G, H), lambda i: (0, 0))
_AGG_SPEC = pl.BlockSpec((1, RB, H),
                         lambda i: (i // HALF_BLOCKS, i % HALF_BLOCKS, 0))

_tc0 = pl.pallas_call(
    _tc0_body,
    grid=(NB,),
    in_specs=[_row_spec(DIN), _B3_SPEC, _full2((DIN, H)), _full2((DIN, H))],
    out_specs=[_row_spec(H), _POOL_SPEC],
    out_shape=[jax.ShapeDtypeStruct((N, H), jnp.float32),
               jax.ShapeDtypeStruct((G, H), jnp.float32)],
    scratch_shapes=[pltpu.VMEM((G, H), jnp.float32)],
)

_VEC_SPECS = [_full2((1, H))] * 2 + [_full2((H, H))] + [_full2((1, H))] * 4

_tc_mlp = pl.pallas_call(
    functools.partial(_tc_mlp_body, has_next=True),
    grid=(NB,),
    in_specs=[_row_spec(H), _AGG_SPEC, _B3_SPEC] + _VEC_SPECS
             + [_full2((H, H))],
    out_specs=[_row_spec(H), _POOL_SPEC],
    out_shape=[jax.ShapeDtypeStruct((N, H), jnp.float32),
               jax.ShapeDtypeStruct((G, H), jnp.float32)],
    scratch_shapes=[pltpu.VMEM((G, H), jnp.float32)],
)

_tc_last = pl.pallas_call(
    functools.partial(_tc_mlp_body, has_next=False),
    grid=(NB,),
    in_specs=[_row_spec(H), _AGG_SPEC, _B3_SPEC] + _VEC_SPECS
             + [_full2((H, H))],
    out_specs=[_row_spec(H), _POOL_SPEC],
    out_shape=[jax.ShapeDtypeStruct((N, H), jnp.float32),
               jax.ShapeDtypeStruct((G, H), jnp.float32)],
    scratch_shapes=[pltpu.VMEM((G, H), jnp.float32)],
)

_tc_final = pl.pallas_call(
    _tc_final_body,
    out_shape=jax.ShapeDtypeStruct((G, H), jnp.float32),
)


def kernel(x, edge_index, batch, W1_0, W1_rest, b1, W2, b2, bng1, bnb1,
           bng2, bnb2, bng3, bnb3, P0, Pb0, P_rest, Pb_rest):
  f32 = jnp.float32
  r = 1.0 / jnp.sqrt(jnp.asarray(1.0 + BN_EPS, f32))
  a1 = bng1 * r
  d1 = b1 * a1 + bnb1
  a2 = bng2 * r
  d2 = b2 * a2 + bnb2
  a3 = bng3 * r
  c3 = bnb3

  src = edge_index[0]
  dst = edge_index[1]
  pad = E_PAD - E
  src2d = jnp.concatenate([src, jnp.zeros((pad,), jnp.int32)]).reshape(
      E_PAD // 128, 128)
  dst2d = jnp.concatenate([dst, jnp.full((pad,), N, jnp.int32)]).reshape(
      E_PAD // 128, 128)
  zero_rows = jnp.zeros((ZROWS, H), f32)
  batch3 = batch.reshape(NB, 1, RB)

  y, pxp = _tc0(x, batch3, W1_0, P0)

  bsrc_f, bdst_f, cnt = _get_sc_part()(src2d, dst2d)
  bsrc = bsrc_f.reshape(NC, CAPR, 128)
  bdst = bdst_f.reshape(NC, CAPR, 128)

  sc_agg = _get_sc_agg()
  pooled = []
  for l in range(NLAYERS):
    agg = sc_agg(y, bsrc, bdst, cnt, zero_rows)
    vecs = [a1[l].reshape(1, H), d1[l].reshape(1, H), W2[l],
            a2[l].reshape(1, H), d2[l].reshape(1, H),
            a3[l].reshape(1, H), c3[l].reshape(1, H)]
    if l < NLAYERS - 1:
      wn = W1_rest[l]
      y, pool_l = _tc_mlp(y, agg, batch3, *vecs, wn)
    else:
      _, pool_l = _tc_last(y, agg, batch3, *vecs, jnp.zeros((H, H), f32))
    pooled.append(pool_l)

  cvec = (Pb0 + Pb_rest.sum(axis=0)).reshape(1, H)
  score = _tc_final(pxp, *pooled, P_rest, cvec)
  return (score,) + tuple(pooled)


# speedup vs baseline: 12.3360x; 1.0010x over previous
import functools

import jax
import jax.numpy as jnp
from jax import lax
from jax.experimental import pallas as pl
from jax.experimental.pallas import tpu as pltpu
from jax.experimental.pallas import tpu_sc as plsc

N = 100000
E = 1600000
DIN = 33
H = 32
G = 512
NLAYERS = 5
BN_EPS = 1e-5

NC = 2
NS = 16
HALF = N // NC
ACC_ROWS = 50176
ZROWS = ACC_ROWS // NS
SUB = 4
CHUNK_E = SUB * 128
ASUB = 2
E_PAD = 1605632
EPR = E_PAD // 128
SHARE_ROWS = EPR // NS
IN_CHUNKS = SHARE_ROWS // 8
FLUSH = 1024
CAPR = NS * (SHARE_ROWS + 8)
PAD_DST = N

RB = 2000
NB = N // RB
HALF_BLOCKS = HALF // RB


def _sc_part_body(src_hbm, dst_hbm, bsrc_hbm, bdst_hbm, cnt_hbm,
                  in_src, in_dst, stage_src, stage_dst,
                  flush_src, flush_dst, cnt_v, cnt_smem, sem_f, sem_ld):
  c = lax.axis_index("c")
  s = lax.axis_index("s")
  coff = c * HALF

  @pl.when(s == 0)
  def _():
    cnt_smem[0] = 0
  plsc.subcore_barrier()

  def wait_flush():
    pltpu.make_async_copy(flush_src, bsrc_hbm.at[c, pl.ds(0, FLUSH)],
                          sem_f).wait()
    pltpu.make_async_copy(flush_dst, bdst_hbm.at[c, pl.ds(0, FLUSH)],
                          sem_f).wait()

  def chunk(k, carry):
    w, flushed = carry
    row0 = s * SHARE_ROWS + k * 8
    pltpu.async_copy(src_hbm.at[pl.ds(row0, 8)], in_src, sem_ld)
    pltpu.async_copy(dst_hbm.at[pl.ds(row0, 8)], in_dst, sem_ld)
    pltpu.make_async_copy(src_hbm.at[pl.ds(row0, 8)], in_src, sem_ld).wait()
    pltpu.make_async_copy(dst_hbm.at[pl.ds(row0, 8)], in_dst, sem_ld).wait()
    for z in range(64):
      zr, zc = z // 8, z % 8
      srcv = in_src[zr, pl.ds(zc * 16, 16)]
      dstv = in_dst[zr, pl.ds(zc * 16, 16)]
      m = (dstv >= coff) & (dstv < coff + HALF)
      nkeep = jnp.max(plsc.all_reduce_population_count(m))
      plsc.store_compressed(stage_src.at[pl.ds(w, 16)], srcv, mask=m)
      plsc.store_compressed(stage_dst.at[pl.ds(w, 16)], dstv, mask=m)
      w = w + nkeep
    do_flush = w >= FLUSH

    @pl.when(do_flush)
    def _():
      @pl.when(flushed == 1)
      def _():
        wait_flush()
      for q in range(FLUSH // 16):
        flush_src[pl.ds(q * 16, 16)] = stage_src[pl.ds(q * 16, 16)]
        flush_dst[pl.ds(q * 16, 16)] = stage_dst[pl.ds(q * 16, 16)]
      off = plsc.fetch_and_add(cnt_smem.at[0], FLUSH, subcore_id=0)
      off = pl.multiple_of(off, FLUSH)
      pltpu.async_copy(flush_src, bsrc_hbm.at[c, pl.ds(off, FLUSH)], sem_f)
      pltpu.async_copy(flush_dst, bdst_hbm.at[c, pl.ds(off, FLUSH)], sem_f)
      for q in range(FLUSH // 16):
        tshift_s = stage_src[pl.ds(FLUSH + q * 16, 16)]
        tshift_d = stage_dst[pl.ds(FLUSH + q * 16, 16)]
        stage_src[pl.ds(q * 16, 16)] = tshift_s
        stage_dst[pl.ds(q * 16, 16)] = tshift_d

    flushed = jnp.where(do_flush, 1, flushed)
    w = jnp.where(do_flush, w - FLUSH, w)
    return (w, flushed)

  w, flushed = lax.fori_loop(0, IN_CHUNKS, chunk, (jnp.int32(0), jnp.int32(0)))

  @pl.when(flushed == 1)
  def _():
    wait_flush()

  @pl.when(w > 0)
  def _():
    for q in range(FLUSH // 16):
      idxv = q * 16 + lax.iota(jnp.int32, 16)
      keep = idxv < w
      tsrc = jnp.where(keep, stage_src[pl.ds(q * 16, 16)], 0)
      tdst = jnp.where(keep, stage_dst[pl.ds(q * 16, 16)], PAD_DST)
      flush_src[pl.ds(q * 16, 16)] = tsrc
      flush_dst[pl.ds(q * 16, 16)] = tdst
    off = plsc.fetch_and_add(cnt_smem.at[0], FLUSH, subcore_id=0)
    off = pl.multiple_of(off, FLUSH)
    pltpu.async_copy(flush_src, bsrc_hbm.at[c, pl.ds(off, FLUSH)], sem_f)
    pltpu.async_copy(flush_dst, bdst_hbm.at[c, pl.ds(off, FLUSH)], sem_f)
    wait_flush()

  plsc.subcore_barrier()

  @pl.when(s == 0)
  def _():
    total = cnt_smem[0]
    cnt_v[pl.ds(0, 16)] = jnp.full((16,), total, jnp.int32)
    pltpu.async_copy(cnt_v, cnt_hbm.at[c], sem_f)
    pltpu.make_async_copy(cnt_v, cnt_hbm.at[c], sem_f).wait()


@functools.lru_cache(maxsize=1)
def _get_sc_part():
  return pl.kernel(
      _sc_part_body,
      out_type=(
          jax.ShapeDtypeStruct((NC, CAPR * 128), jnp.int32),
          jax.ShapeDtypeStruct((NC, CAPR * 128), jnp.int32),
          jax.ShapeDtypeStruct((NC, 16), jnp.int32),
      ),
      mesh=plsc.VectorSubcoreMesh(core_axis_name="c", subcore_axis_name="s"),
      compiler_params=pltpu.CompilerParams(use_tc_tiling_on_sc=False,
                                           needs_layout_passes=False),
      scratch_types=[
          pltpu.VMEM((8, 128), jnp.int32),
          pltpu.VMEM((8, 128), jnp.int32),
          pltpu.VMEM((2 * FLUSH + 16,), jnp.int32),
          pltpu.VMEM((2 * FLUSH + 16,), jnp.int32),
          pltpu.VMEM((FLUSH,), jnp.int32),
          pltpu.VMEM((FLUSH,), jnp.int32),
          pltpu.VMEM((16,), jnp.int32),
          pltpu.SMEM((8,), jnp.int32),
          pltpu.SemaphoreType.DMA,
          pltpu.SemaphoreType.DMA,
      ],
  )


def _sc_agg_body(y_hbm, bsrc_hbm, bdst_hbm, cnt_hbm, zero_hbm, agg_hbm,
                 acc, src_v, dst_v, lidx_v, rows_v, cnt_v,
                 sem_g, sem_s0, sem_s1, sem_i0, sem_i1):
  c = lax.axis_index("c")
  s = lax.axis_index("s")
  sem_s = (sem_s0, sem_s1)
  sem_i = (sem_i0, sem_i1)
  pltpu.sync_copy(cnt_hbm.at[c], cnt_v)
  pltpu.sync_copy(zero_hbm, acc.at[pl.ds(s * ZROWS, ZROWS)])
  n_edges = jnp.max(cnt_v[pl.ds(0, 16)])
  n_chunks = n_edges // (ASUB * 128)
  trips = jnp.maximum((n_chunks - s + NS - 1) // NS, 0)
  plsc.subcore_barrier()
  coff = c * HALF

  def wait_scatters(b):
    for j in range(ASUB):
      pltpu.make_async_copy(rows_v.at[b, j], acc.at[lidx_v.at[b, j]],
                            sem_s[b]).wait()

  def fire_idx(i, b):
    row0 = (s + i * NS) * ASUB
    pltpu.async_copy(bsrc_hbm.at[c, pl.ds(row0, ASUB)], src_v.at[b],
                     sem_i[b])
    pltpu.async_copy(bdst_hbm.at[c, pl.ds(row0, ASUB)], dst_v.at[b],
                     sem_i[b])

  def wait_idx(i, b):
    row0 = (s + i * NS) * ASUB
    pltpu.make_async_copy(bsrc_hbm.at[c, pl.ds(row0, ASUB)], src_v.at[b],
                          sem_i[b]).wait()
    pltpu.make_async_copy(bdst_hbm.at[c, pl.ds(row0, ASUB)], dst_v.at[b],
                          sem_i[b]).wait()

  @pl.when(trips >= 1)
  def _():
    fire_idx(0, 0)

  @pl.when(trips >= 2)
  def _():
    fire_idx(1, 1)

  def body(p, carry):
    for b in range(2):
      i = 2 * p + b

      @pl.when(i < trips)
      def _():
        @pl.when(i >= 2)
        def _():
          wait_scatters(b)
        wait_idx(i, b)
        gathers = [
            pltpu.async_copy(y_hbm.at[src_v.at[b, j]], rows_v.at[b, j],
                             sem_g)
            for j in range(ASUB)
        ]
        for j in range(ASUB):
          for i2 in range(128 // 16):
            v = dst_v[b, j, pl.ds(i2 * 16, 16)]
            lv = v - coff
            lv = jnp.where((lv < 0) | (lv >= HALF), HALF, lv)
            lidx_v[b, j, pl.ds(i2 * 16, 16)] = lv
        for g in gathers:
          g.wait()
        for j in range(ASUB):
          pltpu.async_copy(rows_v.at[b, j], acc.at[lidx_v.at[b, j]],
                           sem_s[b], add=True)

        @pl.when(i + 2 < trips)
        def _():
          fire_idx(i + 2, b)
    return carry

  lax.fori_loop(0, (trips + 1) // 2, body, 0)

  @pl.when(trips >= 1)
  def _():
    wait_scatters(0)

  @pl.when(trips >= 2)
  def _():
    wait_scatters(1)
  plsc.subcore_barrier()
  pltpu.sync_copy(acc.at[pl.ds(s * ZROWS, ZROWS)],
                  agg_hbm.at[c, pl.ds(s * ZROWS, ZROWS)])


@functools.lru_cache(maxsize=1)
def _get_sc_agg():
  return pl.kernel(
      _sc_agg_body,
      out_type=jax.ShapeDtypeStruct((NC, ACC_ROWS, H), jnp.float32),
      mesh=plsc.VectorSubcoreMesh(core_axis_name="c", subcore_axis_name="s"),
      compiler_params=pltpu.CompilerParams(use_tc_tiling_on_sc=False,
                                           needs_layout_passes=False),
      scratch_types=[
          pltpu.VMEM_SHARED((ACC_ROWS, H), jnp.float32),
          pltpu.VMEM((2, ASUB, 128), jnp.int32),
          pltpu.VMEM((2, ASUB, 128), jnp.int32),
          pltpu.VMEM((2, ASUB, 128), jnp.int32),
          pltpu.VMEM((2, ASUB, 128, H), jnp.float32),
          pltpu.VMEM((16,), jnp.int32),
          pltpu.SemaphoreType.DMA,
          pltpu.SemaphoreType.DMA,
          pltpu.SemaphoreType.DMA,
          pltpu.SemaphoreType.DMA,
          pltpu.SemaphoreType.DMA,
      ],
  )


def _onehot_pool(ids, rows):
  oh = (lax.broadcasted_iota(jnp.int32, (RB, G), 1) == ids[:, None])
  return lax.dot_general(oh.astype(jnp.float32), rows,
                         (((0,), (0,)), ((), ())),
                         preferred_element_type=jnp.float32)


def _tc0_body(x_ref, b3_ref, w1_ref, p0_ref, y_ref, pool_ref, pacc):
  i = pl.program_id(0)
  x = x_ref[...]
  y_ref[...] = jnp.dot(x, w1_ref[...], preferred_element_type=jnp.float32)
  xp = jnp.dot(x, p0_ref[...], preferred_element_type=jnp.float32)

  @pl.when(i == 0)
  def _():
    pacc[...] = jnp.zeros_like(pacc)

  pacc[...] += _onehot_pool(b3_ref[0, 0, :], xp)

  @pl.when(i == NB - 1)
  def _():
    pool_ref[...] = pacc[...]


def _tc_mlp_body(y_ref, agg_ref, b3_ref, a1, d1, w2, a2, d2, a3, c3, wn,
                 yout_ref, pool_ref, pacc, *, has_next):
  i = pl.program_id(0)
  t = y_ref[...] + agg_ref[0]
  u1 = jnp.maximum(t * a1[...] + d1[...], 0.0)
  z2 = jnp.dot(u1, w2[...], preferred_element_type=jnp.float32)
  u2 = jnp.maximum(z2 * a2[...] + d2[...], 0.0)
  h = jnp.maximum(u2 * a3[...] + c3[...], 0.0)
  if has_next:
    yout_ref[...] = jnp.dot(h, wn[...], preferred_element_type=jnp.float32)

  @pl.when(i == 0)
  def _():
    pacc[...] = jnp.zeros_like(pacc)

  pacc[...] += _onehot_pool(b3_ref[0, 0, :], h)

  @pl.when(i == NB - 1)
  def _():
    pool_ref[...] = pacc[...]


def _tc_final_body(pxp_ref, p0r, p1r, p2r, p3r, p4r, prest_ref, cvec_ref,
                   score_ref):
  acc = pxp_ref[...] + cvec_ref[...]
  for l, pr in enumerate((p0r, p1r, p2r, p3r, p4r)):
    acc = acc + jnp.dot(pr[...], prest_ref[l],
                        preferred_element_type=jnp.float32)
  score_ref[...] = acc


def _row_spec(w):
  return pl.BlockSpec((RB, w), lambda i: (i, 0))


def _full2(shape):
  return pl.BlockSpec(shape, lambda i: tuple(0 for _ in shape))


_B3_SPEC = pl.BlockSpec((1, 1, RB), lambda i: (i, 0, 0))
_POOL_SPEC = pl.BlockSpec((G, H), lambda i: (0, 0))
_AGG_SPEC = pl.BlockSpec((1, RB, H),
                         lambda i: (i // HALF_BLOCKS, i % HALF_BLOCKS, 0))

_tc0 = pl.pallas_call(
    _tc0_body,
    grid=(NB,),
    in_specs=[_row_spec(DIN), _B3_SPEC, _full2((DIN, H)), _full2((DIN, H))],
    out_specs=[_row_spec(H), _POOL_SPEC],
    out_shape=[jax.ShapeDtypeStruct((N, H), jnp.float32),
               jax.ShapeDtypeStruct((G, H), jnp.float32)],
    scratch_shapes=[pltpu.VMEM((G, H), jnp.float32)],
)

_VEC_SPECS = [_full2((1, H))] * 2 + [_full2((H, H))] + [_full2((1, H))] * 4

_tc_mlp = pl.pallas_call(
    functools.partial(_tc_mlp_body, has_next=True),
    grid=(NB,),
    in_specs=[_row_spec(H), _AGG_SPEC, _B3_SPEC] + _VEC_SPECS
             + [_full2((H, H))],
    out_specs=[_row_spec(H), _POOL_SPEC],
    out_shape=[jax.ShapeDtypeStruct((N, H), jnp.float32),
               jax.ShapeDtypeStruct((G, H), jnp.float32)],
    scratch_shapes=[pltpu.VMEM((G, H), jnp.float32)],
)

_tc_last = pl.pallas_call(
    functools.partial(_tc_mlp_body, has_next=False),
    grid=(NB,),
    in_specs=[_row_spec(H), _AGG_SPEC, _B3_SPEC] + _VEC_SPECS
             + [_full2((H, H))],
    out_specs=[_row_spec(H), _POOL_SPEC],
    out_shape=[jax.ShapeDtypeStruct((N, H), jnp.float32),
               jax.ShapeDtypeStruct((G, H), jnp.float32)],
    scratch_shapes=[pltpu.VMEM((G, H), jnp.float32)],
)

_tc_final = pl.pallas_call(
    _tc_final_body,
    out_shape=jax.ShapeDtypeStruct((G, H), jnp.float32),
)


def kernel(x, edge_index, batch, W1_0, W1_rest, b1, W2, b2, bng1, bnb1,
           bng2, bnb2, bng3, bnb3, P0, Pb0, P_rest, Pb_rest):
  f32 = jnp.float32
  r = 1.0 / jnp.sqrt(jnp.asarray(1.0 + BN_EPS, f32))
  a1 = bng1 * r
  d1 = b1 * a1 + bnb1
  a2 = bng2 * r
  d2 = b2 * a2 + bnb2
  a3 = bng3 * r
  c3 = bnb3

  src = edge_index[0]
  dst = edge_index[1]
  pad = E_PAD - E
  src2d = jnp.concatenate([src, jnp.zeros((pad,), jnp.int32)]).reshape(
      E_PAD // 128, 128)
  dst2d = jnp.concatenate([dst, jnp.full((pad,), N, jnp.int32)]).reshape(
      E_PAD // 128, 128)
  zero_rows = jnp.zeros((ZROWS, H), f32)
  batch3 = batch.reshape(NB, 1, RB)

  y, pxp = _tc0(x, batch3, W1_0, P0)

  bsrc_f, bdst_f, cnt = _get_sc_part()(src2d, dst2d)
  bsrc = bsrc_f.reshape(NC, CAPR, 128)
  bdst = bdst_f.reshape(NC, CAPR, 128)

  sc_agg = _get_sc_agg()
  pooled = []
  for l in range(NLAYERS):
    agg = sc_agg(y, bsrc, bdst, cnt, zero_rows)
    vecs = [a1[l].reshape(1, H), d1[l].reshape(1, H), W2[l],
            a2[l].reshape(1, H), d2[l].reshape(1, H),
            a3[l].reshape(1, H), c3[l].reshape(1, H)]
    if l < NLAYERS - 1:
      wn = W1_rest[l]
      y, pool_l = _tc_mlp(y, agg, batch3, *vecs, wn)
    else:
      _, pool_l = _tc_last(y, agg, batch3, *vecs, jnp.zeros((H, H), f32))
    pooled.append(pool_l)

  cvec = (Pb0 + Pb_rest.sum(axis=0)).reshape(1, H)
  score = _tc_final(pxp, *pooled, P_rest, cvec)
  return (score,) + tuple(pooled)


# speedup vs baseline: 12.4224x; 1.0070x over previous
import functools

import jax
import jax.numpy as jnp
from jax import lax
from jax.experimental import pallas as pl
from jax.experimental.pallas import tpu as pltpu
from jax.experimental.pallas import tpu_sc as plsc

N = 100000
E = 1600000
DIN = 33
H = 32
G = 512
NLAYERS = 5
BN_EPS = 1e-5

NC = 2
NS = 16
HALF = N // NC
ACC_ROWS = 50176
ZROWS = ACC_ROWS // NS
SUB = 4
CHUNK_E = SUB * 128
ASUB = 2
E_PAD = 1605632
EPR = E_PAD // 128
SHARE_ROWS = EPR // NS
IN_CHUNKS = SHARE_ROWS // 8
FLUSH = 1024
CAPR = NS * (SHARE_ROWS + 8)
PAD_DST = N

RB = 2000
NB = N // RB
HALF_BLOCKS = HALF // RB


def _sc_part_body(src_hbm, dst_hbm, bsrc_hbm, bdst_hbm, cnt_hbm,
                  in_src, in_dst, stage_src, stage_dst,
                  flush_src, flush_dst, cnt_v, cnt_smem, sem_f, sem_ld):
  c = lax.axis_index("c")
  s = lax.axis_index("s")
  coff = c * HALF

  @pl.when(s == 0)
  def _():
    cnt_smem[0] = 0
  plsc.subcore_barrier()

  def wait_flush():
    pltpu.make_async_copy(flush_src, bsrc_hbm.at[c, pl.ds(0, FLUSH)],
                          sem_f).wait()
    pltpu.make_async_copy(flush_dst, bdst_hbm.at[c, pl.ds(0, FLUSH)],
                          sem_f).wait()

  def chunk(k, carry):
    w, flushed = carry
    row0 = s * SHARE_ROWS + k * 8
    pltpu.async_copy(src_hbm.at[pl.ds(row0, 8)], in_src, sem_ld)
    pltpu.async_copy(dst_hbm.at[pl.ds(row0, 8)], in_dst, sem_ld)
    pltpu.make_async_copy(src_hbm.at[pl.ds(row0, 8)], in_src, sem_ld).wait()
    pltpu.make_async_copy(dst_hbm.at[pl.ds(row0, 8)], in_dst, sem_ld).wait()
    for z in range(64):
      zr, zc = z // 8, z % 8
      srcv = in_src[zr, pl.ds(zc * 16, 16)]
      dstv = in_dst[zr, pl.ds(zc * 16, 16)]
      m = (dstv >= coff) & (dstv < coff + HALF)
      nkeep = jnp.max(plsc.all_reduce_population_count(m))
      plsc.store_compressed(stage_src.at[pl.ds(w, 16)], srcv, mask=m)
      plsc.store_compressed(stage_dst.at[pl.ds(w, 16)], dstv, mask=m)
      w = w + nkeep
    do_flush = w >= FLUSH

    @pl.when(do_flush)
    def _():
      @pl.when(flushed == 1)
      def _():
        wait_flush()
      for q in range(FLUSH // 16):
        flush_src[pl.ds(q * 16, 16)] = stage_src[pl.ds(q * 16, 16)]
        flush_dst[pl.ds(q * 16, 16)] = stage_dst[pl.ds(q * 16, 16)]
      off = plsc.fetch_and_add(cnt_smem.at[0], FLUSH, subcore_id=0)
      off = pl.multiple_of(off, FLUSH)
      pltpu.async_copy(flush_src, bsrc_hbm.at[c, pl.ds(off, FLUSH)], sem_f)
      pltpu.async_copy(flush_dst, bdst_hbm.at[c, pl.ds(off, FLUSH)], sem_f)
      for q in range(FLUSH // 16):
        tshift_s = stage_src[pl.ds(FLUSH + q * 16, 16)]
        tshift_d = stage_dst[pl.ds(FLUSH + q * 16, 16)]
        stage_src[pl.ds(q * 16, 16)] = tshift_s
        stage_dst[pl.ds(q * 16, 16)] = tshift_d

    flushed = jnp.where(do_flush, 1, flushed)
    w = jnp.where(do_flush, w - FLUSH, w)
    return (w, flushed)

  w, flushed = lax.fori_loop(0, IN_CHUNKS, chunk, (jnp.int32(0), jnp.int32(0)))

  @pl.when(flushed == 1)
  def _():
    wait_flush()

  @pl.when(w > 0)
  def _():
    for q in range(FLUSH // 16):
      idxv = q * 16 + lax.iota(jnp.int32, 16)
      keep = idxv < w
      tsrc = jnp.where(keep, stage_src[pl.ds(q * 16, 16)], 0)
      tdst = jnp.where(keep, stage_dst[pl.ds(q * 16, 16)], PAD_DST)
      flush_src[pl.ds(q * 16, 16)] = tsrc
      flush_dst[pl.ds(q * 16, 16)] = tdst
    off = plsc.fetch_and_add(cnt_smem.at[0], FLUSH, subcore_id=0)
    off = pl.multiple_of(off, FLUSH)
    pltpu.async_copy(flush_src, bsrc_hbm.at[c, pl.ds(off, FLUSH)], sem_f)
    pltpu.async_copy(flush_dst, bdst_hbm.at[c, pl.ds(off, FLUSH)], sem_f)
    wait_flush()

  plsc.subcore_barrier()

  @pl.when(s == 0)
  def _():
    total = cnt_smem[0]
    cnt_v[pl.ds(0, 16)] = jnp.full((16,), total, jnp.int32)
    pltpu.async_copy(cnt_v, cnt_hbm.at[c], sem_f)
    pltpu.make_async_copy(cnt_v, cnt_hbm.at[c], sem_f).wait()


@functools.lru_cache(maxsize=1)
def _get_sc_part():
  return pl.kernel(
      _sc_part_body,
      out_type=(
          jax.ShapeDtypeStruct((NC, CAPR * 128), jnp.int32),
          jax.ShapeDtypeStruct((NC, CAPR * 128), jnp.int32),
          jax.ShapeDtypeStruct((NC, 16), jnp.int32),
      ),
      mesh=plsc.VectorSubcoreMesh(core_axis_name="c", subcore_axis_name="s"),
      compiler_params=pltpu.CompilerParams(use_tc_tiling_on_sc=False,
                                           needs_layout_passes=False),
      scratch_types=[
          pltpu.VMEM((8, 128), jnp.int32),
          pltpu.VMEM((8, 128), jnp.int32),
          pltpu.VMEM((2 * FLUSH + 16,), jnp.int32),
          pltpu.VMEM((2 * FLUSH + 16,), jnp.int32),
          pltpu.VMEM((FLUSH,), jnp.int32),
          pltpu.VMEM((FLUSH,), jnp.int32),
          pltpu.VMEM((16,), jnp.int32),
          pltpu.SMEM((8,), jnp.int32),
          pltpu.SemaphoreType.DMA,
          pltpu.SemaphoreType.DMA,
      ],
  )


def _sc_agg_body(y_hbm, bsrc_hbm, bdst_hbm, cnt_hbm, zero_hbm, agg_hbm,
                 acc, src_v, dst_v, lidx_v, rows_v, cnt_v,
                 sem_g, sem_s0, sem_s1, sem_i0, sem_i1):
  c = lax.axis_index("c")
  s = lax.axis_index("s")
  sem_s = (sem_s0, sem_s1)
  sem_i = (sem_i0, sem_i1)
  pltpu.sync_copy(cnt_hbm.at[c], cnt_v)
  pltpu.sync_copy(zero_hbm, acc.at[pl.ds(s * ZROWS, ZROWS)])
  n_edges = jnp.max(cnt_v[pl.ds(0, 16)])
  n_chunks = n_edges // (ASUB * 128)
  trips = jnp.maximum((n_chunks - s + NS - 1) // NS, 0)
  plsc.subcore_barrier()
  coff = c * HALF

  def wait_scatters(b):
    for j in range(ASUB):
      pltpu.make_async_copy(rows_v.at[b, j], acc.at[lidx_v.at[b, j]],
                            sem_s[b]).wait()

  def fire_idx(i, b):
    row0 = (s + i * NS) * ASUB
    pltpu.async_copy(bsrc_hbm.at[c, pl.ds(row0, ASUB)], src_v.at[b],
                     sem_i[b])
    pltpu.async_copy(bdst_hbm.at[c, pl.ds(row0, ASUB)], dst_v.at[b],
                     sem_i[b])

  def wait_idx(i, b):
    row0 = (s + i * NS) * ASUB
    pltpu.make_async_copy(bsrc_hbm.at[c, pl.ds(row0, ASUB)], src_v.at[b],
                          sem_i[b]).wait()
    pltpu.make_async_copy(bdst_hbm.at[c, pl.ds(row0, ASUB)], dst_v.at[b],
                          sem_i[b]).wait()

  @pl.when(trips >= 1)
  def _():
    fire_idx(0, 0)

  @pl.when(trips >= 2)
  def _():
    fire_idx(1, 1)

  def body(p, carry):
    for b in range(2):
      i = 2 * p + b

      @pl.when(i < trips)
      def _():
        @pl.when(i >= 2)
        def _():
          wait_scatters(b)
        wait_idx(i, b)
        gathers = [
            pltpu.async_copy(y_hbm.at[src_v.at[b, j]], rows_v.at[b, j],
                             sem_g)
            for j in range(ASUB)
        ]
        for j in range(ASUB):
          for i2 in range(128 // 16):
            v = dst_v[b, j, pl.ds(i2 * 16, 16)]
            lv = v - coff
            lv = jnp.where((lv < 0) | (lv >= HALF), HALF, lv)
            lidx_v[b, j, pl.ds(i2 * 16, 16)] = lv
        for g in gathers:
          g.wait()
        for j in range(ASUB):
          pltpu.async_copy(rows_v.at[b, j], acc.at[lidx_v.at[b, j]],
                           sem_s[b], add=True)

        @pl.when(i + 2 < trips)
        def _():
          fire_idx(i + 2, b)
    return carry

  lax.fori_loop(0, (trips + 1) // 2, body, 0)

  @pl.when(trips >= 1)
  def _():
    wait_scatters(0)

  @pl.when(trips >= 2)
  def _():
    wait_scatters(1)
  plsc.subcore_barrier()
  pltpu.sync_copy(acc.at[pl.ds(s * ZROWS, ZROWS)],
                  agg_hbm.at[c, pl.ds(s * ZROWS, ZROWS)])


@functools.lru_cache(maxsize=1)
def _get_sc_agg():
  return pl.kernel(
      _sc_agg_body,
      out_type=jax.ShapeDtypeStruct((NC, ACC_ROWS, H), jnp.float32),
      mesh=plsc.VectorSubcoreMesh(core_axis_name="c", subcore_axis_name="s"),
      compiler_params=pltpu.CompilerParams(use_tc_tiling_on_sc=False,
                                           needs_layout_passes=False),
      scratch_types=[
          pltpu.VMEM_SHARED((ACC_ROWS, H), jnp.float32),
          pltpu.VMEM((2, ASUB, 128), jnp.int32),
          pltpu.VMEM((2, ASUB, 128), jnp.int32),
          pltpu.VMEM((2, ASUB, 128), jnp.int32),
          pltpu.VMEM((2, ASUB, 128, H), jnp.float32),
          pltpu.VMEM((16,), jnp.int32),
          pltpu.SemaphoreType.DMA,
          pltpu.SemaphoreType.DMA,
          pltpu.SemaphoreType.DMA,
          pltpu.SemaphoreType.DMA,
          pltpu.SemaphoreType.DMA,
      ],
  )


def _onehot_pool(ids, rows):
  oh = (lax.broadcasted_iota(jnp.int32, (RB, G), 1) == ids[:, None])
  return lax.dot_general(oh.astype(jnp.float32), rows,
                         (((0,), (0,)), ((), ())),
                         preferred_element_type=jnp.float32)


def _tc0_body(x_ref, b3_ref, w1_ref, p0_ref, y_ref, pool_ref, pacc):
  i = pl.program_id(0)
  x = x_ref[...]
  y_ref[...] = jnp.dot(x, w1_ref[...], preferred_element_type=jnp.float32)
  xp = jnp.dot(x, p0_ref[...], preferred_element_type=jnp.float32)

  @pl.when(i == 0)
  def _():
    pacc[...] = jnp.zeros_like(pacc)

  pacc[...] += _onehot_pool(b3_ref[0, 0, :], xp)

  @pl.when(i == NB - 1)
  def _():
    pool_ref[...] = pacc[...]


def _mlp_h(y_ref, agg_ref, a1, d1, w2, a2, d2, a3, c3):
  t = y_ref[...] + agg_ref[0]
  u1 = jnp.maximum(t * a1[...] + d1[...], 0.0)
  z2 = jnp.dot(u1, w2[...], preferred_element_type=jnp.float32)
  u2 = jnp.maximum(z2 * a2[...] + d2[...], 0.0)
  return jnp.maximum(u2 * a3[...] + c3[...], 0.0)


def _tc_y_body(y_ref, agg_ref, a1, d1, w2, a2, d2, a3, c3, wn, yout_ref):
  h = _mlp_h(y_ref, agg_ref, a1, d1, w2, a2, d2, a3, c3)
  yout_ref[...] = jnp.dot(h, wn[...], preferred_element_type=jnp.float32)


def _tc_pool_body(y_ref, agg_ref, b3_ref, a1, d1, w2, a2, d2, a3, c3,
                  pool_ref, pacc):
  i = pl.program_id(0)
  h = _mlp_h(y_ref, agg_ref, a1, d1, w2, a2, d2, a3, c3)

  @pl.when(i == 0)
  def _():
    pacc[...] = jnp.zeros_like(pacc)

  pacc[...] += _onehot_pool(b3_ref[0, 0, :], h)

  @pl.when(i == NB - 1)
  def _():
    pool_ref[...] = pacc[...]


def _tc_final_body(pxp_ref, p0r, p1r, p2r, p3r, p4r, prest_ref, cvec_ref,
                   score_ref):
  acc = pxp_ref[...] + cvec_ref[...]
  for l, pr in enumerate((p0r, p1r, p2r, p3r, p4r)):
    acc = acc + jnp.dot(pr[...], prest_ref[l],
                        preferred_element_type=jnp.float32)
  score_ref[...] = acc


def _row_spec(w):
  return pl.BlockSpec((RB, w), lambda i: (i, 0))


def _full2(shape):
  return pl.BlockSpec(shape, lambda i: tuple(0 for _ in shape))


_B3_SPEC = pl.BlockSpec((1, 1, RB), lambda i: (i, 0, 0))
_POOL_SPEC = pl.BlockSpec((G, H), lambda i: (0, 0))
_AGG_SPEC = pl.BlockSpec((1, RB, H),
                         lambda i: (i // HALF_BLOCKS, i % HALF_BLOCKS, 0))

_tc0 = pl.pallas_call(
    _tc0_body,
    grid=(NB,),
    in_specs=[_row_spec(DIN), _B3_SPEC, _full2((DIN, H)), _full2((DIN, H))],
    out_specs=[_row_spec(H), _POOL_SPEC],
    out_shape=[jax.ShapeDtypeStruct((N, H), jnp.float32),
               jax.ShapeDtypeStruct((G, H), jnp.float32)],
    scratch_shapes=[pltpu.VMEM((G, H), jnp.float32)],
)

_VEC_SPECS = [_full2((1, H))] * 2 + [_full2((H, H))] + [_full2((1, H))] * 4

_tc_y = pl.pallas_call(
    _tc_y_body,
    grid=(NB,),
    in_specs=[_row_spec(H), _AGG_SPEC] + _VEC_SPECS + [_full2((H, H))],
    out_specs=_row_spec(H),
    out_shape=jax.ShapeDtypeStruct((N, H), jnp.float32),
)

_tc_pool = pl.pallas_call(
    _tc_pool_body,
    grid=(NB,),
    in_specs=[_row_spec(H), _AGG_SPEC, _B3_SPEC] + _VEC_SPECS,
    out_specs=_POOL_SPEC,
    out_shape=jax.ShapeDtypeStruct((G, H), jnp.float32),
    scratch_shapes=[pltpu.VMEM((G, H), jnp.float32)],
)

_tc_final = pl.pallas_call(
    _tc_final_body,
    out_shape=jax.ShapeDtypeStruct((G, H), jnp.float32),
)


def kernel(x, edge_index, batch, W1_0, W1_rest, b1, W2, b2, bng1, bnb1,
           bng2, bnb2, bng3, bnb3, P0, Pb0, P_rest, Pb_rest):
  f32 = jnp.float32
  r = 1.0 / jnp.sqrt(jnp.asarray(1.0 + BN_EPS, f32))
  a1 = bng1 * r
  d1 = b1 * a1 + bnb1
  a2 = bng2 * r
  d2 = b2 * a2 + bnb2
  a3 = bng3 * r
  c3 = bnb3

  src = edge_index[0]
  dst = edge_index[1]
  pad = E_PAD - E
  src2d = jnp.concatenate([src, jnp.zeros((pad,), jnp.int32)]).reshape(
      E_PAD // 128, 128)
  dst2d = jnp.concatenate([dst, jnp.full((pad,), N, jnp.int32)]).reshape(
      E_PAD // 128, 128)
  zero_rows = jnp.zeros((ZROWS, H), f32)
  batch3 = batch.reshape(NB, 1, RB)

  y, pxp = _tc0(x, batch3, W1_0, P0)

  bsrc_f, bdst_f, cnt = _get_sc_part()(src2d, dst2d)
  bsrc = bsrc_f.reshape(NC, CAPR, 128)
  bdst = bdst_f.reshape(NC, CAPR, 128)

  sc_agg = _get_sc_agg()
  pooled = []
  for l in range(NLAYERS):
    agg = sc_agg(y, bsrc, bdst, cnt, zero_rows)
    vecs = [a1[l].reshape(1, H), d1[l].reshape(1, H), W2[l],
            a2[l].reshape(1, H), d2[l].reshape(1, H),
            a3[l].reshape(1, H), c3[l].reshape(1, H)]
    y_next = None
    if l < NLAYERS - 1:
      y_next = _tc_y(y, agg, *vecs, W1_rest[l])
    pool_l = _tc_pool(y, agg, batch3, *vecs)
    pooled.append(pool_l)
    if y_next is not None:
      y = y_next

  cvec = (Pb0 + Pb_rest.sum(axis=0)).reshape(1, H)
  score = _tc_final(pxp, *pooled, P_rest, cvec)
  return (score,) + tuple(pooled)
